# Initial kernel scaffold; baseline (speedup 1.0000x reference)
import jax
import jax.numpy as jnp
from jax.experimental import pallas as pl


def kernel(tags, feature_counts, ln1_g, ln1_b, W1, b1, ln2_g, ln2_b, W2, b2, W3, b3, W4, b4, W5, b5):
    raise NotImplementedError("write your pallas kernel here")


# speedup vs baseline: 2.9558x; 2.9558x over previous
import functools

import jax
import jax.numpy as jnp
from jax import lax
from jax.experimental import pallas as pl
from jax.experimental.pallas import tpu as pltpu
from jax.experimental.pallas import tpu_sc as plsc

B, F, T = 256, 8, 20
NUM_TAGS = 20000
H = 256
D = NUM_TAGS + 1
CP = 20480
KB = 2048
NKB = CP // KB
DUMMY = 20500
BUF = 20544
TPAD = 256
ROWS_PER_W = B // 32


@functools.lru_cache(maxsize=1)
def _sc_hist_fn():
    mesh = plsc.VectorSubcoreMesh(core_axis_name="c", subcore_axis_name="s")

    @functools.partial(
        pl.kernel,
        out_type=jax.ShapeDtypeStruct((B, CP), jnp.float32),
        mesh=mesh,
        compiler_params=pltpu.CompilerParams(use_tc_tiling_on_sc=False,
                                             needs_layout_passes=False),
        scratch_types=[
            pltpu.VMEM((BUF,), jnp.int32),
            pltpu.VMEM((BUF,), jnp.float32),
            pltpu.VMEM((TPAD,), jnp.int32),
        ],
    )
    def _sc_hist(tags_hbm, counts_hbm, bits, cnts, trow):
        wid = lax.axis_index("s") * 2 + lax.axis_index("c")
        zero_i = jnp.zeros((16,), jnp.int32)
        zero_f = jnp.zeros((16,), jnp.float32)

        def zinit(i, carry):
            bits[pl.ds(i * 16, 16)] = zero_i
            cnts[pl.ds(i * 16, 16)] = zero_f
            return carry

        lax.fori_loop(0, BUF // 16, zinit, 0)

        def do_row(r, carry):
            row = wid * ROWS_PER_W + r
            pltpu.sync_copy(tags_hbm.at[row], trow)
            for f in range(F):
                bit = jnp.full((16,), 1 << f, jnp.int32)
                for ch in range(2):
                    idx = trow[pl.ds(f * 32 + ch * 16, 16)]
                    old = plsc.load_gather(bits, [idx])
                    plsc.store_scatter(bits, [idx], old | bit)
            for ch in range(TPAD // 16):
                idx = trow[pl.ds(ch * 16, 16)]
                v = plsc.load_gather(bits, [idx])
                v = v - ((v >> jnp.full((16,), 1, jnp.int32))
                         & jnp.full((16,), 0x55, jnp.int32))
                v = ((v & jnp.full((16,), 0x33, jnp.int32))
                     + ((v >> jnp.full((16,), 2, jnp.int32))
                        & jnp.full((16,), 0x33, jnp.int32)))
                v = ((v + (v >> jnp.full((16,), 4, jnp.int32)))
                     & jnp.full((16,), 0x0F, jnp.int32))
                plsc.store_scatter(cnts, [idx], v.astype(jnp.float32))
            pltpu.sync_copy(cnts.at[pl.ds(0, CP)], counts_hbm.at[row])
            for ch in range(TPAD // 16):
                idx = trow[pl.ds(ch * 16, 16)]
                plsc.store_scatter(bits, [idx], zero_i)
                plsc.store_scatter(cnts, [idx], zero_f)
            return carry

        lax.fori_loop(0, ROWS_PER_W, do_row, 0)

    return _sc_hist


def _gelu(x):
    return x * 0.5 * (1.0 + lax.erf(x * (2.0 ** -0.5)))


def _stats_body(cnt_ref, m_ref, r_ref, s_acc, q_acc):
    k = pl.program_id(0)

    @pl.when(k == 0)
    def _():
        s_acc[...] = jnp.zeros_like(s_acc)
        q_acc[...] = jnp.zeros_like(q_acc)

    cb = cnt_ref[...]
    s_acc[...] += jnp.sum(cb, axis=1, keepdims=True)
    q_acc[...] += jnp.sum(cb * cb, axis=1, keepdims=True)

    @pl.when(k == NKB - 1)
    def _():
        m = s_acc[...] / NUM_TAGS
        var = q_acc[...] / NUM_TAGS - m * m
        m_ref[...] = m
        r_ref[...] = lax.rsqrt(var + 1e-5)


def _stats(counts):
    return pl.pallas_call(
        _stats_body,
        grid=(NKB,),
        in_specs=[pl.BlockSpec((B, KB), lambda k: (0, k))],
        out_specs=[
            pl.BlockSpec((B, 1), lambda k: (0, 0)),
            pl.BlockSpec((B, 1), lambda k: (0, 0)),
        ],
        out_shape=[
            jax.ShapeDtypeStruct((B, 1), jnp.float32),
            jax.ShapeDtypeStruct((B, 1), jnp.float32),
        ],
        scratch_shapes=[
            pltpu.VMEM((B, 1), jnp.float32),
            pltpu.VMEM((B, 1), jnp.float32),
        ],
    )(counts)


def _enc_body(cnt_ref, w1_ref, gs_ref, bs_ref, m_ref, r_ref, fc_ref,
              b1_ref, g2_ref, b2l_ref, w2_ref, b2_ref, w3_ref, b3_ref,
              w4_ref, b4_ref, y_ref, enc_ref, d_ref, hacc):
    k = pl.program_id(0)
    cb = cnt_ref[...]
    yb = (cb - m_ref[...]) * r_ref[...] * gs_ref[...] + bs_ref[...]
    col = lax.broadcasted_iota(jnp.int32, (B, KB), 1) + k * KB
    yb = jnp.where(col < D, yb, 0.0)
    yb = jnp.where(col == 0, fc_ref[...] * 0.01, yb)
    y_ref[...] = yb

    @pl.when(k == 0)
    def _():
        hacc[...] = jnp.zeros_like(hacc)

    rowi = lax.broadcasted_iota(jnp.int32, (KB, 4 * H), 0) + k * KB
    w1b = jnp.where(rowi < D, w1_ref[...], 0.0)
    hacc[...] += jnp.dot(yb, w1b, preferred_element_type=jnp.float32)

    @pl.when(k == NKB - 1)
    def _():
        h = _gelu(hacc[...] + b1_ref[...])
        mu = jnp.mean(h, axis=-1, keepdims=True)
        var = jnp.mean((h - mu) ** 2, axis=-1, keepdims=True)
        h = (h - mu) * lax.rsqrt(var + 1e-5) * g2_ref[...] + b2l_ref[...]
        h = _gelu(jnp.dot(h, w2_ref[...], preferred_element_type=jnp.float32)
                  + b2_ref[...])
        e = jnp.dot(h, w3_ref[...], preferred_element_type=jnp.float32) + b3_ref[...]
        enc_ref[...] = e
        d_ref[...] = _gelu(jnp.dot(e, w4_ref[...], preferred_element_type=jnp.float32)
                           + b4_ref[...])


def _encoder(counts, W1, gs, bs, m, r, fc, b1, g2, b2l, W2, b2, W3, b3, W4, b4):
    fixed = lambda k: (0, 0)
    return pl.pallas_call(
        _enc_body,
        grid=(NKB,),
        in_specs=[
            pl.BlockSpec((B, KB), lambda k: (0, k)),
            pl.BlockSpec((KB, 4 * H), lambda k: (k, 0)),
            pl.BlockSpec((1, KB), lambda k: (0, k)),
            pl.BlockSpec((1, KB), lambda k: (0, k)),
            pl.BlockSpec((B, 1), fixed),
            pl.BlockSpec((B, 1), fixed),
            pl.BlockSpec((B, 1), fixed),
            pl.BlockSpec((1, 4 * H), fixed),
            pl.BlockSpec((1, 4 * H), fixed),
            pl.BlockSpec((1, 4 * H), fixed),
            pl.BlockSpec((4 * H, 4 * H), fixed),
            pl.BlockSpec((1, 4 * H), fixed),
            pl.BlockSpec((4 * H, H), fixed),
            pl.BlockSpec((1, H), fixed),
            pl.BlockSpec((H, 4 * H), fixed),
            pl.BlockSpec((1, 4 * H), fixed),
        ],
        out_specs=[
            pl.BlockSpec((B, KB), lambda k: (0, k)),
            pl.BlockSpec((B, H), fixed),
            pl.BlockSpec((B, 4 * H), fixed),
        ],
        out_shape=[
            jax.ShapeDtypeStruct((B, D), jnp.float32),
            jax.ShapeDtypeStruct((B, H), jnp.float32),
            jax.ShapeDtypeStruct((B, 4 * H), jnp.float32),
        ],
        scratch_shapes=[pltpu.VMEM((B, 4 * H), jnp.float32)],
    )(counts, W1, gs, bs, m, r, fc, b1, g2, b2l, W2, b2, W3, b3, W4, b4)


def _dec_body(d_ref, w5_ref, b5_ref, dec_ref):
    dec_ref[...] = (jnp.dot(d_ref[...], w5_ref[...],
                            preferred_element_type=jnp.float32) + b5_ref[...])


def _decoder(d, W5, b5):
    return pl.pallas_call(
        _dec_body,
        grid=(NKB,),
        in_specs=[
            pl.BlockSpec((B, 4 * H), lambda k: (0, 0)),
            pl.BlockSpec((4 * H, KB), lambda k: (0, k)),
            pl.BlockSpec((1, KB), lambda k: (0, k)),
        ],
        out_specs=[pl.BlockSpec((B, KB), lambda k: (0, k))],
        out_shape=[jax.ShapeDtypeStruct((B, D), jnp.float32)],
    )(d, W5, b5)


def kernel(tags, feature_counts, ln1_g, ln1_b, W1, b1, ln2_g, ln2_b, W2, b2,
           W3, b3, W4, b4, W5, b5):
    tags_p = jnp.pad(tags.astype(jnp.int32) + 1, ((0, 0), (0, 0), (0, 12)),
                     constant_values=DUMMY).reshape(B, TPAD)
    counts = _sc_hist_fn()(tags_p)
    gs = jnp.pad(ln1_g, (1, CP - 1 - NUM_TAGS)).reshape(1, CP)
    bs = jnp.pad(ln1_b, (1, CP - 1 - NUM_TAGS)).reshape(1, CP)
    m, r = _stats(counts)
    y, enc, d = _encoder(
        counts, W1, gs, bs, m, r, feature_counts.reshape(B, 1),
        b1.reshape(1, -1), ln2_g.reshape(1, -1), ln2_b.reshape(1, -1),
        W2, b2.reshape(1, -1), W3, b3.reshape(1, -1), W4, b4.reshape(1, -1))
    (dec,) = _decoder(d, W5, b5.reshape(1, -1))
    return (y, enc, dec)
